# baseline (speedup 1.0000x reference)
import functools

import jax
import jax.numpy as jnp
from jax import lax
from jax.experimental import pallas as pl
from jax.experimental.pallas import tpu as pltpu
from jax.experimental.pallas import tpu_sc as plsc

_L = 16
_NBUF = 2
_SPG = 4


def _build_kernel(S, B0, V, D, W):
    info = plsc.get_sparse_core_info()
    nc = info.num_cores
    nw = nc * info.num_subcores
    assert B0 % nw == 0 and W == B0 // nw
    G = S // _SPG
    assert S % _SPG == 0 and G % _NBUF == 0
    mesh = plsc.VectorSubcoreMesh(core_axis_name="c", subcore_axis_name="s")

    @functools.partial(
        pl.kernel,
        mesh=mesh,
        out_type=jax.ShapeDtypeStruct((S, D, B0), jnp.float32),
        scratch_types=[
            pltpu.VMEM((S, W), jnp.int32),
            [pltpu.VMEM((_SPG * W, D), jnp.float32) for _ in range(_NBUF)],
            [pltpu.VMEM((D, W + 1), jnp.float32) for _ in range(_NBUF)],
            [pltpu.SemaphoreType.DMA for _ in range(_NBUF)],
            [pltpu.SemaphoreType.DMA for _ in range(_NBUF)],
        ],
        compiler_params=pltpu.CompilerParams(
            use_tc_tiling_on_sc=False, needs_layout_passes=False
        ),
    )
    def k(xp_hbm, table_hbm, out_hbm, idx_all, rows, blks, gsems, wsems):
        wid = lax.axis_index("s") * nc + lax.axis_index("c")
        base = wid * W
        pltpu.sync_copy(xp_hbm.at[:, pl.ds(base, W)], idx_all)

        vm = (V // 512) * 512

        def remap(s, c):
            for q in range(W // _L):
                v = idx_all[s, pl.ds(q * _L, _L)]
                j = v & 511
                k_main = (v & -512) + 2 * (j & 255) + ((j >> 8) & 1)
                jt = v - vm
                k_tail = vm + 2 * (jt & 31) + ((jt >> 5) & 1)
                k = jnp.where(v < vm, k_main, k_tail)
                idx_all[s, pl.ds(q * _L, _L)] = k
            return c

        lax.fori_loop(0, S, remap, 0)

        def g_start(g, b):
            for j in range(_SPG):
                s = g * _SPG + j
                pltpu.async_copy(
                    table_hbm.at[idx_all.at[s]],
                    rows[b].at[pl.ds(j * W, W), :],
                    gsems[b],
                )

        def g_wait(g, b):
            for j in range(_SPG):
                s = g * _SPG + j
                pltpu.make_async_copy(
                    table_hbm.at[idx_all.at[s]],
                    rows[b].at[pl.ds(j * W, W), :],
                    gsems[b],
                ).wait()

        def w_start(s, wb):
            pltpu.async_copy(
                blks[wb].at[:, pl.ds(0, W)],
                out_hbm.at[s, :, pl.ds(base, W)],
                wsems[wb],
            )

        def w_wait(s, wb):
            pltpu.make_async_copy(
                blks[wb].at[:, pl.ds(0, W)],
                out_hbm.at[s, :, pl.ds(base, W)],
                wsems[wb],
            ).wait()

        def transpose(gb, j, wb):
            rv, bv = rows[gb], blks[wb]
            off = j * W

            def ti(i2, c):
                i0 = i2 * 2
                for di in range(2):
                    i = i0 + di
                    cvec = jnp.zeros((_L,), jnp.int32) + i
                    for p in range(D // _L):
                        rvec = lax.iota(jnp.int32, _L) + (p * _L)
                        vals = rv[off + i, pl.ds(p * _L, _L)]
                        plsc.store_scatter(bv, [rvec, cvec], vals)
                return c

            lax.fori_loop(0, W // 2, ti, 0)

        def process_group(g, gb, first):
            for j in range(_SPG):
                s = g * _SPG + j
                wb = j % 2
                if not first:
                    w_wait(s - 2, wb)
                elif j >= 2:
                    w_wait(s - 2, wb)
                transpose(gb, j, wb)
                w_start(s, wb)

        for b in range(_NBUF):
            g_start(b, b)
        g_wait(0, 0)
        process_group(0, 0, True)
        g_start(_NBUF, 0)

        def body(g, carry):
            for b in range(_NBUF):
                gg = g * _NBUF + b
                g_wait(gg, b)
                process_group(gg, b, False)
                g_start(gg + _NBUF, b)
            return carry

        g_wait(1, 1)
        process_group(1, 1, False)
        g_start(1 + _NBUF, 1)

        lax.fori_loop(1, G // _NBUF - 1, body, 0)

        for b in range(_NBUF):
            gg = G - _NBUF + b
            g_wait(gg, b)
            process_group(gg, b, False)
        for s in (S - 2, S - 1):
            w_wait(s, s % 2)

    return k


def _tc_transpose(V, D, C=512):
    Vm = (V // C) * C
    grid = Vm // C
    rows_out = V * D // 128
    blk_rows = C * D // 128
    Ct = V - Vm
    tail_rows = Ct * D // 128

    def mk_body(cw):
        def body(in_ref, out_ref):
            eye = jnp.eye(D, dtype=jnp.float32)
            t = jax.lax.dot_general(
                in_ref[...], eye, (((0,), (0,)), ((), ())),
                preferred_element_type=jnp.float32,
            )
            h = cw // 2
            out_ref[...] = jnp.concatenate([t[:h, :], t[h:, :]], axis=1)

        return body

    def run(emb_t):
        main = pl.pallas_call(
            mk_body(C),
            grid=(grid,),
            in_specs=[pl.BlockSpec((D, C), lambda i: (0, i))],
            out_specs=pl.BlockSpec((blk_rows, 128), lambda i: (i, 0)),
            out_shape=jax.ShapeDtypeStruct((rows_out, 128), jnp.float32),
        )(emb_t)

        def tail_body(prev_ref, in_ref, out_ref):
            del prev_ref
            eye = jnp.eye(D, dtype=jnp.float32)
            t = jax.lax.dot_general(
                in_ref[...], eye, (((0,), (0,)), ((), ())),
                preferred_element_type=jnp.float32,
            )
            h = Ct // 2
            out_ref[...] = jnp.concatenate(
                [t[128 - Ct : 128 - Ct + h, :], t[128 - h :, :]], axis=1
            )

        emb_tail = lax.slice(emb_t, (0, V - 128), (D, V))
        return pl.pallas_call(
            tail_body,
            grid=(1,),
            in_specs=[
                pl.BlockSpec(memory_space=pltpu.MemorySpace.HBM),
                pl.BlockSpec((D, 128), lambda i: (0, 0)),
            ],
            out_specs=pl.BlockSpec(
                (tail_rows, 128), lambda i: (Vm * D // 128 // tail_rows, 0)
            ),
            out_shape=jax.ShapeDtypeStruct((rows_out, 128), jnp.float32),
            input_output_aliases={0: 0},
        )(main, emb_tail)

    return run


def kernel(x, emb_weight):
    B0, S = x.shape
    V, D = emb_weight.shape
    x_p = x.T
    table_lin = _tc_transpose(V, D)(emb_weight.T)
    table_2d = jnp.reshape(table_lin, (V, D))
    out_p = _build_kernel(S, B0, V, D, B0 // 32)(x_p.astype(jnp.int32), table_2d)
    return jnp.transpose(out_p, (2, 0, 1))


# speedup vs baseline: 1.9901x; 1.9901x over previous
import functools

import jax
import jax.numpy as jnp
from jax import lax
from jax.experimental import pallas as pl
from jax.experimental.pallas import tpu as pltpu
from jax.experimental.pallas import tpu_sc as plsc

_L = 16
_NBUF = 2
_SPG = 4


def _build_kernel(S, B0, V, D, W):
    info = plsc.get_sparse_core_info()
    nc = info.num_cores
    nw = nc * info.num_subcores
    assert B0 % nw == 0 and W == B0 // nw
    G = S // _SPG
    assert S % _SPG == 0 and G % _NBUF == 0
    mesh = plsc.VectorSubcoreMesh(core_axis_name="c", subcore_axis_name="s")

    @functools.partial(
        pl.kernel,
        mesh=mesh,
        out_type=jax.ShapeDtypeStruct((S, D, B0), jnp.float32),
        scratch_types=[
            pltpu.VMEM((S, W), jnp.int32),
            [pltpu.VMEM((_SPG * W, D), jnp.float32) for _ in range(_NBUF)],
            [pltpu.VMEM((D, W + 1), jnp.float32) for _ in range(_NBUF)],
            [pltpu.SemaphoreType.DMA for _ in range(_NBUF)],
            [pltpu.SemaphoreType.DMA for _ in range(_NBUF)],
        ],
        compiler_params=pltpu.CompilerParams(
            use_tc_tiling_on_sc=False, needs_layout_passes=False
        ),
    )
    def k(xp_hbm, table_hbm, out_hbm, idx_all, rows, blks, gsems, wsems):
        wid = lax.axis_index("s") * nc + lax.axis_index("c")
        base = wid * W
        pltpu.sync_copy(xp_hbm.at[:, pl.ds(base, W)], idx_all)

        vm = (V // 512) * 512

        def remap(s, c):
            for q in range(W // _L):
                v = idx_all[s, pl.ds(q * _L, _L)]
                j = v & 511
                k_main = (v & -512) + 2 * (j & 255) + ((j >> 8) & 1)
                jt = v - vm
                k_tail = vm + 2 * (jt & 31) + ((jt >> 5) & 1)
                k = jnp.where(v < vm, k_main, k_tail)
                idx_all[s, pl.ds(q * _L, _L)] = k
            return c

        lax.fori_loop(0, S, remap, 0)

        def g_start(g, b):
            for j in range(_SPG):
                s = g * _SPG + j
                pltpu.async_copy(
                    table_hbm.at[idx_all.at[s]],
                    rows[b].at[pl.ds(j * W, W), :],
                    gsems[b],
                )

        def g_wait(g, b):
            for j in range(_SPG):
                s = g * _SPG + j
                pltpu.make_async_copy(
                    table_hbm.at[idx_all.at[s]],
                    rows[b].at[pl.ds(j * W, W), :],
                    gsems[b],
                ).wait()

        def w_start(s, wb):
            pltpu.async_copy(
                blks[wb].at[:, pl.ds(0, W)],
                out_hbm.at[s, :, pl.ds(base, W)],
                wsems[wb],
            )

        def w_wait(s, wb):
            pltpu.make_async_copy(
                blks[wb].at[:, pl.ds(0, W)],
                out_hbm.at[s, :, pl.ds(base, W)],
                wsems[wb],
            ).wait()

        def transpose(gb, j, wb):
            rv, bv = rows[gb], blks[wb]
            off = j * W

            def ti(i2, c):
                i0 = i2 * 2
                for di in range(2):
                    i = i0 + di
                    cvec = jnp.zeros((_L,), jnp.int32) + i
                    for p in range(D // _L):
                        rvec = lax.iota(jnp.int32, _L) + (p * _L)
                        vals = rv[off + i, pl.ds(p * _L, _L)]
                        plsc.store_scatter(bv, [rvec, cvec], vals)
                return c

            lax.fori_loop(0, W // 2, ti, 0)

        def process_group(g, gb, first):
            for j in range(_SPG):
                s = g * _SPG + j
                wb = j % 2
                if not first:
                    w_wait(s - 2, wb)
                elif j >= 2:
                    w_wait(s - 2, wb)
                transpose(gb, j, wb)
                w_start(s, wb)

        for b in range(_NBUF):
            g_start(b, b)
        g_wait(0, 0)
        process_group(0, 0, True)
        g_start(_NBUF, 0)

        def body(g, carry):
            for b in range(_NBUF):
                gg = g * _NBUF + b
                g_wait(gg, b)
                process_group(gg, b, False)
                g_start(gg + _NBUF, b)
            return carry

        g_wait(1, 1)
        process_group(1, 1, False)
        g_start(1 + _NBUF, 1)

        lax.fori_loop(1, G // _NBUF - 1, body, 0)

        for b in range(_NBUF):
            gg = G - _NBUF + b
            g_wait(gg, b)
            process_group(gg, b, False)
        for s in (S - 2, S - 1):
            w_wait(s, s % 2)

    return k


def _tc_transpose(V, D, C=512):
    K = 9
    Vm = (V // C) * C
    grid = Vm // (C * K)
    rows_out = V * D // 128
    blk_rows = C * D // 128
    Ct = V - Vm
    tail_rows = Ct * D // 128
    h = C // 2

    def body(in_ref, out_ref):
        for t in range(K):
            sub = in_ref[:, C * t : C * (t + 1)].T
            out_ref[blk_rows * t : blk_rows * (t + 1), :] = jnp.concatenate(
                [sub[:h, :], sub[h:, :]], axis=1
            )

    def run(emb_t):
        main = pl.pallas_call(
            body,
            grid=(grid,),
            in_specs=[pl.BlockSpec((D, C * K), lambda i: (0, i))],
            out_specs=pl.BlockSpec((blk_rows * K, 128), lambda i: (i, 0)),
            out_shape=jax.ShapeDtypeStruct((rows_out, 128), jnp.float32),
        )(emb_t)

        def tail_body(prev_ref, in_ref, out_ref):
            del prev_ref
            t = in_ref[...].T
            ht = Ct // 2
            out_ref[...] = jnp.concatenate(
                [t[128 - Ct : 128 - Ct + ht, :], t[128 - ht :, :]], axis=1
            )

        emb_tail = lax.slice(emb_t, (0, V - 128), (D, V))
        return pl.pallas_call(
            tail_body,
            grid=(1,),
            in_specs=[
                pl.BlockSpec(memory_space=pltpu.MemorySpace.HBM),
                pl.BlockSpec((D, 128), lambda i: (0, 0)),
            ],
            out_specs=pl.BlockSpec(
                (tail_rows, 128), lambda i: (Vm * D // 128 // tail_rows, 0)
            ),
            out_shape=jax.ShapeDtypeStruct((rows_out, 128), jnp.float32),
            input_output_aliases={0: 0},
        )(main, emb_tail)

    return run


def kernel(x, emb_weight):
    B0, S = x.shape
    V, D = emb_weight.shape
    x_p = x.T
    table_lin = _tc_transpose(V, D)(emb_weight.T)
    table_2d = jnp.reshape(table_lin, (V, D))
    out_p = _build_kernel(S, B0, V, D, B0 // 32)(x_p.astype(jnp.int32), table_2d)
    return jnp.transpose(out_p, (2, 0, 1))


# speedup vs baseline: 2.6633x; 1.3383x over previous
import functools

import jax
import jax.numpy as jnp
from jax import lax
from jax.experimental import pallas as pl
from jax.experimental.pallas import tpu as pltpu
from jax.experimental.pallas import tpu_sc as plsc

_L = 16
_NBUF = 2
_SPG = 4


def _build_kernel(S, B0, V, D, W):
    info = plsc.get_sparse_core_info()
    nc = info.num_cores
    nw = nc * info.num_subcores
    assert B0 % nw == 0 and W == B0 // nw
    G = S // _SPG
    assert S % _SPG == 0 and G % _NBUF == 0
    mesh = plsc.VectorSubcoreMesh(core_axis_name="c", subcore_axis_name="s")

    @functools.partial(
        pl.kernel,
        mesh=mesh,
        out_type=jax.ShapeDtypeStruct((S, D // 8, B0 // 128, 8, 128), jnp.float32),
        scratch_types=[
            pltpu.VMEM((S, W), jnp.int32),
            [pltpu.VMEM((_SPG * W, D), jnp.float32) for _ in range(_NBUF)],
            [pltpu.VMEM((D // 8, 8, 130), jnp.float32) for _ in range(_NBUF)],
            [pltpu.SemaphoreType.DMA for _ in range(_NBUF)],
            [pltpu.SemaphoreType.DMA for _ in range(_NBUF)],
        ],
        compiler_params=pltpu.CompilerParams(
            use_tc_tiling_on_sc=False, needs_layout_passes=False
        ),
    )
    def k(xp_hbm, table_hbm, out_hbm, idx_all, rows, blks, gsems, wsems):
        wid = lax.axis_index("s") * nc + lax.axis_index("c")
        base = wid * W
        pltpu.sync_copy(xp_hbm.at[:, pl.ds(base, W)], idx_all)

        vm = (V // 512) * 512

        def remap(s, c):
            for q in range(W // _L):
                v = idx_all[s, pl.ds(q * _L, _L)]
                j = v & 511
                k_main = (v & -512) + 2 * (j & 255) + ((j >> 8) & 1)
                jt = v - vm
                k_tail = vm + 2 * (jt & 31) + ((jt >> 5) & 1)
                k = jnp.where(v < vm, k_main, k_tail)
                idx_all[s, pl.ds(q * _L, _L)] = k
            return c

        lax.fori_loop(0, S, remap, 0)

        def g_start(g, b):
            for j in range(_SPG):
                s = g * _SPG + j
                pltpu.async_copy(
                    table_hbm.at[idx_all.at[s]],
                    rows[b].at[pl.ds(j * W, W), :],
                    gsems[b],
                )

        def g_wait(g, b):
            for j in range(_SPG):
                s = g * _SPG + j
                pltpu.make_async_copy(
                    table_hbm.at[idx_all.at[s]],
                    rows[b].at[pl.ds(j * W, W), :],
                    gsems[b],
                ).wait()

        def w_start(s, wb):
            pltpu.async_copy(
                blks[wb].at[:, :, pl.ds(0, 128)],
                out_hbm.at[s, :, wid, :, :],
                wsems[wb],
            )

        def w_wait(s, wb):
            pltpu.make_async_copy(
                blks[wb].at[:, :, pl.ds(0, 128)],
                out_hbm.at[s, :, wid, :, :],
                wsems[wb],
            ).wait()

        _r1 = [
            (lax.iota(jnp.int32, _L) + p * _L) >> 3 for p in range(D // _L)
        ]
        _r2 = [
            (lax.iota(jnp.int32, _L) + p * _L) & 7 for p in range(D // _L)
        ]

        def transpose(gb, j, wb):
            rv, bv = rows[gb], blks[wb]
            off = j * W

            def ti(i2, c):
                i0 = i2 * 2
                for di in range(2):
                    i = i0 + di
                    cvec = jnp.zeros((_L,), jnp.int32) + i
                    for p in range(D // _L):
                        vals = rv[off + i, pl.ds(p * _L, _L)]
                        plsc.store_scatter(bv, [_r1[p], _r2[p], cvec], vals)
                return c

            lax.fori_loop(0, W // 2, ti, 0)

        def process_group(g, gb, first):
            for j in range(_SPG):
                s = g * _SPG + j
                wb = j % 2
                if not first:
                    w_wait(s - 2, wb)
                elif j >= 2:
                    w_wait(s - 2, wb)
                transpose(gb, j, wb)
                w_start(s, wb)

        for b in range(_NBUF):
            g_start(b, b)
        g_wait(0, 0)
        process_group(0, 0, True)
        g_start(_NBUF, 0)

        def body(g, carry):
            for b in range(_NBUF):
                gg = g * _NBUF + b
                g_wait(gg, b)
                process_group(gg, b, False)
                g_start(gg + _NBUF, b)
            return carry

        g_wait(1, 1)
        process_group(1, 1, False)
        g_start(1 + _NBUF, 1)

        lax.fori_loop(1, G // _NBUF - 1, body, 0)

        for b in range(_NBUF):
            gg = G - _NBUF + b
            g_wait(gg, b)
            process_group(gg, b, False)
        for s in (S - 2, S - 1):
            w_wait(s, s % 2)

    return k


def _tc_transpose(V, D, C=512):
    K = 9
    Vm = (V // C) * C
    grid = Vm // (C * K)
    rows_out = V * D // 128
    blk_rows = C * D // 128
    Ct = V - Vm
    tail_rows = Ct * D // 128
    h = C // 2

    def body(in_ref, out_ref):
        for t in range(K):
            sub = in_ref[:, C * t : C * (t + 1)].T
            out_ref[blk_rows * t : blk_rows * (t + 1), :] = jnp.concatenate(
                [sub[:h, :], sub[h:, :]], axis=1
            )

    def run(emb_t):
        main = pl.pallas_call(
            body,
            grid=(grid,),
            in_specs=[pl.BlockSpec((D, C * K), lambda i: (0, i))],
            out_specs=pl.BlockSpec((blk_rows * K, 128), lambda i: (i, 0)),
            out_shape=jax.ShapeDtypeStruct((rows_out, 128), jnp.float32),
        )(emb_t)

        def tail_body(prev_ref, in_ref, out_ref):
            del prev_ref
            t = in_ref[...].T
            ht = Ct // 2
            out_ref[...] = jnp.concatenate(
                [t[128 - Ct : 128 - Ct + ht, :], t[128 - ht :, :]], axis=1
            )

        emb_tail = lax.slice(emb_t, (0, V - 128), (D, V))
        return pl.pallas_call(
            tail_body,
            grid=(1,),
            in_specs=[
                pl.BlockSpec(memory_space=pltpu.MemorySpace.HBM),
                pl.BlockSpec((D, 128), lambda i: (0, 0)),
            ],
            out_specs=pl.BlockSpec(
                (tail_rows, 128), lambda i: (Vm * D // 128 // tail_rows, 0)
            ),
            out_shape=jax.ShapeDtypeStruct((rows_out, 128), jnp.float32),
            input_output_aliases={0: 0},
        )(main, emb_tail)

    return run


def kernel(x, emb_weight):
    B0, S = x.shape
    V, D = emb_weight.shape
    x_p = x.T
    table_lin = _tc_transpose(V, D)(emb_weight.T)
    table_2d = jnp.reshape(table_lin, (V, D))
    out5 = _build_kernel(S, B0, V, D, B0 // 32)(x_p.astype(jnp.int32), table_2d)
    return jnp.transpose(out5, (2, 4, 0, 1, 3)).reshape(B0, S, D)


# speedup vs baseline: 2.9364x; 1.1026x over previous
import functools

import jax
import jax.numpy as jnp
from jax import lax
from jax.experimental import pallas as pl
from jax.experimental.pallas import tpu as pltpu
from jax.experimental.pallas import tpu_sc as plsc

_L = 16
_NBUF = 2
_SPG = 4


_NSLOT = 8
_NBLK = 2


def _build_kernel(S, B0, V, D, W):
    info = plsc.get_sparse_core_info()
    nc = info.num_cores
    nw = nc * info.num_subcores
    assert B0 % nw == 0 and W == B0 // nw and S % _NSLOT == 0
    mesh = plsc.VectorSubcoreMesh(core_axis_name="c", subcore_axis_name="s")

    @functools.partial(
        pl.kernel,
        mesh=mesh,
        out_type=jax.ShapeDtypeStruct((S, D // 8, B0 // 128, 8, 128), jnp.float32),
        scratch_types=[
            pltpu.VMEM((S, W), jnp.int32),
            [pltpu.VMEM((W, D), jnp.float32) for _ in range(_NSLOT)],
            [pltpu.VMEM((D // 8, 8, 130), jnp.float32) for _ in range(_NBLK)],
            [pltpu.SemaphoreType.DMA for _ in range(_NSLOT)],
            [pltpu.SemaphoreType.DMA for _ in range(_NBLK)],
        ],
        compiler_params=pltpu.CompilerParams(
            use_tc_tiling_on_sc=False, needs_layout_passes=False
        ),
    )
    def k(xp_hbm, table_hbm, out_hbm, idx_all, rows, blks, gsems, wsems):
        wid = lax.axis_index("s") * nc + lax.axis_index("c")
        base = wid * W
        pltpu.sync_copy(xp_hbm.at[:, pl.ds(base, W)], idx_all)

        vm = (V // 512) * 512

        def remap(s, c):
            for q in range(W // _L):
                v = idx_all[s, pl.ds(q * _L, _L)]
                j = v & 511
                k_main = (v & -512) + 2 * (j & 255) + ((j >> 8) & 1)
                jt = v - vm
                k_tail = vm + 2 * (jt & 31) + ((jt >> 5) & 1)
                k = jnp.where(v < vm, k_main, k_tail)
                idx_all[s, pl.ds(q * _L, _L)] = k
            return c

        lax.fori_loop(0, S, remap, 0)

        def g_start(s, b):
            pltpu.async_copy(table_hbm.at[idx_all.at[s]], rows[b], gsems[b])

        def g_wait(s, b):
            pltpu.make_async_copy(
                table_hbm.at[idx_all.at[s]], rows[b], gsems[b]
            ).wait()

        def w_start(s, wb):
            pltpu.async_copy(
                blks[wb].at[:, :, pl.ds(0, 128)],
                out_hbm.at[s, :, wid, :, :],
                wsems[wb],
            )

        def w_wait(s, wb):
            pltpu.make_async_copy(
                blks[wb].at[:, :, pl.ds(0, 128)],
                out_hbm.at[s, :, wid, :, :],
                wsems[wb],
            ).wait()

        _r1 = [
            (lax.iota(jnp.int32, _L) + p * _L) >> 3 for p in range(D // _L)
        ]
        _r2 = [
            (lax.iota(jnp.int32, _L) + p * _L) & 7 for p in range(D // _L)
        ]

        def transpose(slot, wb):
            rv, bv = rows[slot], blks[wb]

            def ti(i2, c):
                i0 = i2 * 2
                for di in range(2):
                    i = i0 + di
                    cvec = jnp.zeros((_L,), jnp.int32) + i
                    for p in range(D // _L):
                        vals = rv[i, pl.ds(p * _L, _L)]
                        plsc.store_scatter(bv, [_r1[p], _r2[p], cvec], vals)
                return c

            lax.fori_loop(0, W // 2, ti, 0)

        for j in range(_NSLOT):
            g_start(j, j)
        for j in range(_NSLOT):
            g_wait(j, j)
            if j >= _NBLK:
                w_wait(j - _NBLK, j % _NBLK)
            transpose(j, j % _NBLK)
            w_start(j, j % _NBLK)
            g_start(j + _NSLOT, j)

        def body(gg, carry):
            for j in range(_NSLOT):
                s = gg * _NSLOT + j
                g_wait(s, j)
                w_wait(s - _NBLK, j % _NBLK)
                transpose(j, j % _NBLK)
                w_start(s, j % _NBLK)
                g_start(s + _NSLOT, j)
            return carry

        lax.fori_loop(1, S // _NSLOT - 1, body, 0)

        for j in range(_NSLOT):
            s = S - _NSLOT + j
            g_wait(s, j)
            w_wait(s - _NBLK, j % _NBLK)
            transpose(j, j % _NBLK)
            w_start(s, j % _NBLK)
        for j in range(_NBLK):
            w_wait(S - _NBLK + j, j)

    return k


def _tc_transpose(V, D, C=512):
    K = 21
    Vm = (V // C) * C
    grid = Vm // (C * K)
    rows_out = V * D // 128
    blk_rows = C * D // 128
    Ct = V - Vm
    tail_rows = Ct * D // 128
    h = C // 2

    def body(in_ref, out_ref):
        for t in range(K):
            sub = in_ref[:, C * t : C * (t + 1)].T
            out_ref[blk_rows * t : blk_rows * (t + 1), :] = jnp.concatenate(
                [sub[:h, :], sub[h:, :]], axis=1
            )

    def run(emb_t):
        main = pl.pallas_call(
            body,
            grid=(grid,),
            in_specs=[pl.BlockSpec((D, C * K), lambda i: (0, i))],
            out_specs=pl.BlockSpec((blk_rows * K, 128), lambda i: (i, 0)),
            out_shape=jax.ShapeDtypeStruct((rows_out, 128), jnp.float32),
        )(emb_t)

        def tail_body(prev_ref, in_ref, out_ref):
            del prev_ref
            t = in_ref[...].T
            ht = Ct // 2
            out_ref[...] = jnp.concatenate(
                [t[128 - Ct : 128 - Ct + ht, :], t[128 - ht :, :]], axis=1
            )

        emb_tail = lax.slice(emb_t, (0, V - 128), (D, V))
        return pl.pallas_call(
            tail_body,
            grid=(1,),
            in_specs=[
                pl.BlockSpec(memory_space=pltpu.MemorySpace.HBM),
                pl.BlockSpec((D, 128), lambda i: (0, 0)),
            ],
            out_specs=pl.BlockSpec(
                (tail_rows, 128), lambda i: (Vm * D // 128 // tail_rows, 0)
            ),
            out_shape=jax.ShapeDtypeStruct((rows_out, 128), jnp.float32),
            input_output_aliases={0: 0},
        )(main, emb_tail)

    return run


def kernel(x, emb_weight):
    B0, S = x.shape
    V, D = emb_weight.shape
    x_p = x.T
    table_lin = _tc_transpose(V, D)(emb_weight.T)
    table_2d = jnp.reshape(table_lin, (V, D))
    out5 = _build_kernel(S, B0, V, D, B0 // 32)(x_p.astype(jnp.int32), table_2d)
    return jnp.transpose(out5, (2, 4, 0, 1, 3)).reshape(B0, S, D)
